# baseline (speedup 1.0000x reference)
import functools

import jax
import jax.numpy as jnp
from jax import lax
from jax.experimental import pallas as pl
from jax.experimental.pallas import tpu as pltpu
from jax.experimental.pallas import tpu_sc as plsc

N = 10000
E = 320000
D = 128
DG = 16

NC = 2
NS = 16
NW = NC * NS
EPW = E // NW
CHUNK = 80
NCHUNK = EPW // CHUNK
PHA = 63
PHB = NCHUNK - PHA

RPS = 640


def _sc_agg_body(x_hbm, src_hbm, dst_hbm, outf_hbm, outd_hbm,
                 acc_sh, deg_sh, src_v, dst_v, rows0_v, rows1_v, ones_v,
                 g0, g1, s0, s1, d0, d1):
    c = lax.axis_index("c")
    s = lax.axis_index("s")
    wid = c * NS + s

    pltpu.sync_copy(src_hbm.at[wid], src_v)
    pltpu.sync_copy(dst_hbm.at[wid, pl.ds(0, PHA)], dst_v)

    zvec = jnp.zeros((16,), jnp.float32)

    def zero_row(r, _):
        for j in range(D // 16):
            rows0_v[r, pl.ds(j * 16, 16)] = zvec
        ones_v[r, :] = zvec
        return 0

    lax.fori_loop(0, CHUNK, zero_row, 0)
    for z in range(RPS // CHUNK):
        r0 = s * RPS + z * CHUNK

        def zfill(r0=r0):
            pltpu.sync_copy(rows0_v, acc_sh.at[pl.ds(r0, CHUNK)])
            pltpu.sync_copy(ones_v, deg_sh.at[pl.ds(r0, CHUNK)])

        if z < 5:
            zfill()
        else:
            pl.when(s < NS - 1)(zfill)

    ovec = jnp.ones((16,), jnp.float32)

    def ones_row(r, _):
        ones_v[r, :] = ovec
        return 0

    lax.fori_loop(0, CHUNK, ones_row, 0)
    plsc.subcore_barrier()

    def start_gather(k, buf, sem):
        pltpu.async_copy(x_hbm.at[src_v.at[pl.ds(k * CHUNK, CHUNK)]],
                         buf, sem)

    def wait_gather(k, buf, sem):
        pltpu.make_async_copy(x_hbm.at[src_v.at[pl.ds(k * CHUNK, CHUNK)]],
                              buf, sem).wait()

    def start_scatter(kl, buf, sem):
        pltpu.async_copy(buf, acc_sh.at[dst_v.at[kl]], sem, add=True)

    def wait_scatter(kl, buf, sem):
        pltpu.make_async_copy(buf, acc_sh.at[dst_v.at[kl]], sem).wait()

    def start_deg(kl, sem):
        pltpu.async_copy(ones_v, deg_sh.at[dst_v.at[kl]], sem, add=True)

    def wait_deg(kl, sem):
        pltpu.make_async_copy(ones_v, deg_sh.at[dst_v.at[kl]], sem).wait()

    start_gather(0, rows0_v, g0)
    start_gather(1, rows1_v, g1)

    def pair_a(p, _):
        k0 = 2 * p
        wait_gather(k0, rows0_v, g0)
        start_scatter(k0, rows0_v, s0)

        @pl.when(k0 >= 2)
        def _():
            wait_deg(k0 - 2, d0)

        start_deg(k0, d0)
        wait_gather(k0 + 1, rows1_v, g1)
        start_scatter(k0 + 1, rows1_v, s1)

        @pl.when(k0 >= 1)
        def _():
            wait_deg(k0 - 1, d1)

        start_deg(k0 + 1, d1)
        wait_scatter(k0, rows0_v, s0)
        start_gather(k0 + 2, rows0_v, g0)
        wait_scatter(k0 + 1, rows1_v, s1)

        @pl.when(k0 + 3 < PHA)
        def _():
            start_gather(k0 + 3, rows1_v, g1)

        return 0

    lax.fori_loop(0, PHA // 2, pair_a, 0)
    wait_gather(PHA - 1, rows0_v, g0)
    start_scatter(PHA - 1, rows0_v, s0)
    wait_deg(PHA - 3, d0)
    start_deg(PHA - 1, d0)
    wait_scatter(PHA - 1, rows0_v, s0)

    wait_deg(0, d0)
    wait_deg(0, d1)
    pltpu.sync_copy(dst_hbm.at[wid, pl.ds(PHA, PHB)],
                    dst_v.at[pl.ds(0, PHB)])

    start_gather(PHA, rows0_v, g0)
    start_gather(PHA + 1, rows1_v, g1)

    def pair_b(p, _):
        k0 = 2 * p
        wait_gather(PHA + k0, rows0_v, g0)
        start_scatter(k0, rows0_v, s0)

        @pl.when(k0 >= 2)
        def _():
            wait_deg(k0 - 2, d0)

        start_deg(k0, d0)
        wait_gather(PHA + k0 + 1, rows1_v, g1)
        start_scatter(k0 + 1, rows1_v, s1)

        @pl.when(k0 >= 1)
        def _():
            wait_deg(k0 - 1, d1)

        start_deg(k0 + 1, d1)
        wait_scatter(k0, rows0_v, s0)

        @pl.when(k0 + 2 < PHB)
        def _():
            start_gather(PHA + k0 + 2, rows0_v, g0)

        wait_scatter(k0 + 1, rows1_v, s1)

        @pl.when(k0 + 3 < PHB)
        def _():
            start_gather(PHA + k0 + 3, rows1_v, g1)

        return 0

    lax.fori_loop(0, PHB // 2, pair_b, 0)
    wait_deg(0, d0)
    wait_deg(0, d1)
    plsc.subcore_barrier()

    for z in range(RPS // CHUNK):
        r0 = s * RPS + z * CHUNK

        def wb(r0=r0):
            pltpu.sync_copy(acc_sh.at[pl.ds(r0, CHUNK)],
                            outf_hbm.at[c, pl.ds(r0, CHUNK)])
            pltpu.sync_copy(deg_sh.at[pl.ds(r0, CHUNK)],
                            outd_hbm.at[c, pl.ds(r0, CHUNK)])

        if z < 5:
            wb()
        else:
            pl.when(s < NS - 1)(wb)


_sc_agg = functools.partial(
    pl.kernel,
    out_type=(jax.ShapeDtypeStruct((NC, N, D), jnp.float32),
              jax.ShapeDtypeStruct((NC, N, DG), jnp.float32)),
    mesh=plsc.VectorSubcoreMesh(core_axis_name="c", subcore_axis_name="s"),
    scratch_types=[
        pltpu.VMEM_SHARED((N, D), jnp.float32),
        pltpu.VMEM_SHARED((N, DG), jnp.float32),
        pltpu.VMEM((EPW,), jnp.int32),
        pltpu.VMEM((PHA, CHUNK), jnp.int32),
        pltpu.VMEM((CHUNK, D), jnp.float32),
        pltpu.VMEM((CHUNK, D), jnp.float32),
        pltpu.VMEM((CHUNK, DG), jnp.float32),
        pltpu.SemaphoreType.DMA,
        pltpu.SemaphoreType.DMA,
        pltpu.SemaphoreType.DMA,
        pltpu.SemaphoreType.DMA,
        pltpu.SemaphoreType.DMA,
        pltpu.SemaphoreType.DMA,
    ],
    compiler_params=pltpu.CompilerParams(use_tc_tiling_on_sc=False),
)(_sc_agg_body)


BLK = 2000


def _tc_body(p_ref, d_ref, w_ref, o_ref):
    p = p_ref[...]
    dp = d_ref[...]
    t = p[0] + p[1]
    deg = dp[0, :, 0:1] + dp[1, :, 0:1]
    h = t / jnp.clip(deg, 1.0, None)
    o_ref[...] = jnp.dot(h, w_ref[...], preferred_element_type=jnp.float32)


def kernel(x, edge_index, W):
    src = edge_index[0].reshape(NW, EPW)
    dst = edge_index[1].reshape(NW, NCHUNK, CHUNK)
    partial, degp = _sc_agg(x, src, dst)
    z = pl.pallas_call(
        _tc_body,
        grid=(N // BLK,),
        in_specs=[
            pl.BlockSpec((NC, BLK, D), lambda i: (0, i, 0)),
            pl.BlockSpec((NC, BLK, DG), lambda i: (0, i, 0)),
            pl.BlockSpec((D, D), lambda i: (0, 0)),
        ],
        out_specs=pl.BlockSpec((BLK, D), lambda i: (i, 0)),
        out_shape=jax.ShapeDtypeStruct((N, D), jnp.float32),
    )(partial, degp, W)
    return z


# speedup vs baseline: 1.1047x; 1.1047x over previous
import functools

import jax
import jax.numpy as jnp
from jax import lax
from jax.experimental import pallas as pl
from jax.experimental.pallas import tpu as pltpu
from jax.experimental.pallas import tpu_sc as plsc

N = 10000
E = 320000
D = 128
DG = 16

NC = 2
NS = 16
NW = NC * NS
EPW = E // NW
CHUNK = 80
NCHUNK = EPW // CHUNK
PHA = 63
PHB = NCHUNK - PHA

RPS = 640


def _sc_agg_body(x_hbm, src_hbm, dst_hbm, outf_hbm, outd_hbm,
                 acc_sh, deg_sh, src_v, dst_v, rows0_v, rows1_v, ones_v,
                 g0, g1, d0, d1):
    c = lax.axis_index("c")
    s = lax.axis_index("s")
    wid = c * NS + s

    pltpu.sync_copy(src_hbm.at[wid], src_v)
    pltpu.sync_copy(dst_hbm.at[wid, pl.ds(0, PHA)], dst_v)

    zvec = jnp.zeros((16,), jnp.float32)

    def zero_row(r, _):
        for j in range(D // 16):
            rows0_v[r, pl.ds(j * 16, 16)] = zvec
        ones_v[r, :] = zvec
        return 0

    lax.fori_loop(0, CHUNK, zero_row, 0)
    for z in range(RPS // CHUNK):
        r0 = s * RPS + z * CHUNK

        def zfill(r0=r0):
            pltpu.sync_copy(rows0_v, acc_sh.at[pl.ds(r0, CHUNK)])
            pltpu.sync_copy(ones_v, deg_sh.at[pl.ds(r0, CHUNK)])

        if z < 5:
            zfill()
        else:
            pl.when(s < NS - 1)(zfill)

    ovec = jnp.ones((16,), jnp.float32)

    def ones_row(r, _):
        ones_v[r, :] = ovec
        return 0

    lax.fori_loop(0, CHUNK, ones_row, 0)
    plsc.subcore_barrier()

    def start_gather(k, buf, sem):
        pltpu.async_copy(x_hbm.at[src_v.at[pl.ds(k * CHUNK, CHUNK)]],
                         buf, sem)

    def wait_gather(k, buf, sem):
        pltpu.make_async_copy(x_hbm.at[src_v.at[pl.ds(k * CHUNK, CHUNK)]],
                              buf, sem).wait()

    def scatter(kl, buf):
        pltpu.sync_copy(buf, acc_sh.at[dst_v.at[kl]], add=True)

    def start_deg(kl, sem):
        pltpu.async_copy(ones_v, deg_sh.at[dst_v.at[kl]], sem, add=True)

    def wait_deg(kl, sem):
        pltpu.make_async_copy(ones_v, deg_sh.at[dst_v.at[kl]], sem).wait()

    start_gather(0, rows0_v, g0)

    def pair_a(p, _):
        k0 = 2 * p
        start_gather(k0 + 1, rows1_v, g1)
        wait_gather(k0, rows0_v, g0)
        scatter(k0, rows0_v)

        @pl.when(k0 >= 2)
        def _():
            wait_deg(k0 - 2, d0)

        start_deg(k0, d0)
        start_gather(k0 + 2, rows0_v, g0)
        wait_gather(k0 + 1, rows1_v, g1)
        scatter(k0 + 1, rows1_v)

        @pl.when(k0 >= 1)
        def _():
            wait_deg(k0 - 1, d1)

        start_deg(k0 + 1, d1)
        return 0

    lax.fori_loop(0, PHA // 2, pair_a, 0)
    wait_gather(PHA - 1, rows0_v, g0)
    scatter(PHA - 1, rows0_v)
    wait_deg(PHA - 3, d0)
    start_deg(PHA - 1, d0)

    wait_deg(0, d0)
    wait_deg(0, d1)
    pltpu.sync_copy(dst_hbm.at[wid, pl.ds(PHA, PHB)],
                    dst_v.at[pl.ds(0, PHB)])

    start_gather(PHA, rows1_v, g1)

    def pair_b(p, _):
        k0 = 2 * p
        start_gather(PHA + k0 + 1, rows0_v, g0)
        wait_gather(PHA + k0, rows1_v, g1)
        scatter(k0, rows1_v)

        @pl.when(k0 >= 2)
        def _():
            wait_deg(k0 - 2, d0)

        start_deg(k0, d0)

        @pl.when(k0 + 2 < PHB)
        def _():
            start_gather(PHA + k0 + 2, rows1_v, g1)

        wait_gather(PHA + k0 + 1, rows0_v, g0)
        scatter(k0 + 1, rows0_v)

        @pl.when(k0 >= 1)
        def _():
            wait_deg(k0 - 1, d1)

        start_deg(k0 + 1, d1)
        return 0

    lax.fori_loop(0, PHB // 2, pair_b, 0)
    wait_deg(0, d0)
    wait_deg(0, d1)
    plsc.subcore_barrier()

    for z in range(RPS // CHUNK):
        r0 = s * RPS + z * CHUNK

        def wb(r0=r0):
            pltpu.sync_copy(acc_sh.at[pl.ds(r0, CHUNK)],
                            outf_hbm.at[c, pl.ds(r0, CHUNK)])
            pltpu.sync_copy(deg_sh.at[pl.ds(r0, CHUNK)],
                            outd_hbm.at[c, pl.ds(r0, CHUNK)])

        if z < 5:
            wb()
        else:
            pl.when(s < NS - 1)(wb)


_sc_agg = functools.partial(
    pl.kernel,
    out_type=(jax.ShapeDtypeStruct((NC, N, D), jnp.float32),
              jax.ShapeDtypeStruct((NC, N, DG), jnp.float32)),
    mesh=plsc.VectorSubcoreMesh(core_axis_name="c", subcore_axis_name="s"),
    scratch_types=[
        pltpu.VMEM_SHARED((N, D), jnp.float32),
        pltpu.VMEM_SHARED((N, DG), jnp.float32),
        pltpu.VMEM((EPW,), jnp.int32),
        pltpu.VMEM((PHA, CHUNK), jnp.int32),
        pltpu.VMEM((CHUNK, D), jnp.float32),
        pltpu.VMEM((CHUNK, D), jnp.float32),
        pltpu.VMEM((CHUNK, DG), jnp.float32),
        pltpu.SemaphoreType.DMA,
        pltpu.SemaphoreType.DMA,
        pltpu.SemaphoreType.DMA,
        pltpu.SemaphoreType.DMA,
    ],
    compiler_params=pltpu.CompilerParams(use_tc_tiling_on_sc=False),
)(_sc_agg_body)


BLK = 2000


def _tc_body(p_ref, d_ref, w_ref, o_ref):
    p = p_ref[...]
    dp = d_ref[...]
    t = p[0] + p[1]
    deg = dp[0, :, 0:1] + dp[1, :, 0:1]
    h = t / jnp.clip(deg, 1.0, None)
    o_ref[...] = jnp.dot(h, w_ref[...], preferred_element_type=jnp.float32)


def kernel(x, edge_index, W):
    src = edge_index[0].reshape(NW, EPW)
    dst = edge_index[1].reshape(NW, NCHUNK, CHUNK)
    partial, degp = _sc_agg(x, src, dst)
    z = pl.pallas_call(
        _tc_body,
        grid=(N // BLK,),
        in_specs=[
            pl.BlockSpec((NC, BLK, D), lambda i: (0, i, 0)),
            pl.BlockSpec((NC, BLK, DG), lambda i: (0, i, 0)),
            pl.BlockSpec((D, D), lambda i: (0, 0)),
        ],
        out_specs=pl.BlockSpec((BLK, D), lambda i: (i, 0)),
        out_shape=jax.ShapeDtypeStruct((N, D), jnp.float32),
    )(partial, degp, W)
    return z
